# Initial kernel scaffold; baseline (speedup 1.0000x reference)
import jax
import jax.numpy as jnp
from jax.experimental import pallas as pl


def kernel(x, W_vp_o, b_vp_o, W_so, b_so, W_aw, b_aw, W_vp_i, b_vp_i, W_op_i, b_op_i, W_op_o, b_op_o):
    raise NotImplementedError("write your pallas kernel here")


# speedup vs baseline: 88.3789x; 88.3789x over previous
import functools

import jax
import jax.numpy as jnp
from jax import lax
from jax.experimental import pallas as pl
from jax.experimental.pallas import tpu as pltpu
from jax.experimental.pallas import tpu_sc as plsc

B, L, D = 2, 2048, 1024
H, DH, P = 16, 64, 8
HP = H * P
TL = 256
ROWS = B * L
CENTER = L / 2 - 0.5


def _fold_body(wo_ref, wi_ref, bo_ref, bi_ref, wv_ref, bv_ref):
    wv_ref[...] = jnp.dot(wo_ref[...], wi_ref[...],
                          preferred_element_type=jnp.float32)
    bv_ref[...] = jnp.dot(bo_ref[...], wi_ref[...],
                          preferred_element_type=jnp.float32) + bi_ref[...]


def _fold(W_vp_o, W_vp_i, b_vp_o, b_vp_i):
    return pl.pallas_call(
        _fold_body,
        out_shape=(jax.ShapeDtypeStruct((D, D), jnp.float32),
                   jax.ShapeDtypeStruct((1, D), jnp.float32)),
    )(W_vp_o, W_vp_i, b_vp_o.reshape(1, D), b_vp_i.reshape(1, D))


def _stage_a_body(x_ref, w_ref, b_ref, g_ref,
                  val_ref, i0_ref, i1_ref, c0_ref, c1_ref):
    x = x_ref[0]
    acts = jnp.dot(x, w_ref[...], preferred_element_type=jnp.float32) + b_ref[...]
    val_ref[0] = acts[:, :D]
    so_x = acts[:, D:D + HP]
    so_y = acts[:, D + HP:D + 2 * HP]
    lg = acts[:, D + 2 * HP:D + 3 * HP]
    m = jnp.max(lg, axis=-1, keepdims=True)
    e = jnp.exp(lg - m)
    gs = jnp.dot(e, g_ref[...], preferred_element_type=jnp.float32)
    aw = e / gs
    wx = jnp.maximum(0.0, 1.0 - jnp.abs(so_x))
    py = CENTER + so_y
    y0f = jnp.floor(py)
    t = py - y0f
    y0 = y0f.astype(jnp.int32)
    v0 = ((y0 >= 0) & (y0 <= L - 1)).astype(jnp.float32)
    v1 = ((y0 >= -1) & (y0 <= L - 2)).astype(jnp.float32)
    awx = aw * wx
    c0_ref[0] = awx * (1.0 - t) * v0
    c1_ref[0] = awx * t * v1
    y0c = jnp.clip(y0, 0, L - 1)
    y1c = jnp.clip(y0 + 1, 0, L - 1)
    b = pl.program_id(0)
    hcol = lax.broadcasted_iota(jnp.int32, (TL, HP), 1) // P
    base = b * (L * H) + hcol
    i0_ref[0] = base + y0c * H
    i1_ref[0] = base + y1c * H


def _stage_a(x, W_cat, b_cat, G):
    grid = (B, L // TL)
    return pl.pallas_call(
        _stage_a_body,
        grid=grid,
        in_specs=[
            pl.BlockSpec((1, TL, D), lambda b, i: (b, i, 0)),
            pl.BlockSpec((D, D + 3 * HP), lambda b, i: (0, 0)),
            pl.BlockSpec((1, D + 3 * HP), lambda b, i: (0, 0)),
            pl.BlockSpec((HP, HP), lambda b, i: (0, 0)),
        ],
        out_specs=(
            pl.BlockSpec((1, TL, D), lambda b, i: (b, i, 0)),
            pl.BlockSpec((1, TL, HP), lambda b, i: (b, i, 0)),
            pl.BlockSpec((1, TL, HP), lambda b, i: (b, i, 0)),
            pl.BlockSpec((1, TL, HP), lambda b, i: (b, i, 0)),
            pl.BlockSpec((1, TL, HP), lambda b, i: (b, i, 0)),
        ),
        out_shape=(
            jax.ShapeDtypeStruct((B, L, D), jnp.float32),
            jax.ShapeDtypeStruct((B, L, HP), jnp.int32),
            jax.ShapeDtypeStruct((B, L, HP), jnp.int32),
            jax.ShapeDtypeStruct((B, L, HP), jnp.float32),
            jax.ShapeDtypeStruct((B, L, HP), jnp.float32),
        ),
    )(x, W_cat, b_cat, G)


def _sc_info():
    try:
        info = plsc.get_sparse_core_info()
        return info.num_cores, info.num_subcores
    except Exception:
        return 2, 16


def _sc_body(nc, rpw, vtab_hbm, i0_hbm, i1_hbm, c0_hbm, c1_hbm, out_hbm,
             i0_v, i1_v, c0_v, c1_v, g0_v, g1_v, acc_v, sem0, sem1):
    wid = lax.axis_index("s") * nc + lax.axis_index("c")
    base = wid * rpw
    pltpu.sync_copy(i0_hbm.at[pl.ds(base, rpw)], i0_v)
    pltpu.sync_copy(i1_hbm.at[pl.ds(base, rpw)], i1_v)
    pltpu.sync_copy(c0_hbm.at[pl.ds(base, rpw)], c0_v)
    pltpu.sync_copy(c1_hbm.at[pl.ds(base, rpw)], c1_v)

    def row_body(i, carry):
        cp0 = pltpu.async_copy(vtab_hbm.at[i0_v.at[i]], g0_v, sem0)
        cp1 = pltpu.async_copy(vtab_hbm.at[i1_v.at[i]], g1_v, sem1)
        cp0.wait()
        cp1.wait()
        for h2 in range(H // 2):
            cv0 = c0_v[i, pl.ds(h2 * 16, 16)]
            cv1 = c1_v[i, pl.ds(h2 * 16, 16)]
            for sub in range(2):
                h = h2 * 2 + sub
                acc = [jnp.zeros((16,), jnp.float32) for _ in range(4)]
                for p in range(P):
                    r = h * P + p
                    b0 = jnp.full((16,), cv0[sub * P + p], jnp.float32)
                    b1 = jnp.full((16,), cv1[sub * P + p], jnp.float32)
                    for cc in range(4):
                        acc[cc] = (acc[cc]
                                   + b0 * g0_v[r, pl.ds(cc * 16, 16)]
                                   + b1 * g1_v[r, pl.ds(cc * 16, 16)])
                for cc in range(4):
                    acc_v[pl.ds(h * DH + cc * 16, 16)] = acc[cc]
        pltpu.sync_copy(acc_v, out_hbm.at[base + i])
        return carry

    lax.fori_loop(0, rpw, row_body, 0)


def _stage_b(vtab, i0, i1, c0, c1):
    nc, ns = _sc_info()
    nw = nc * ns
    rpw = ROWS // nw
    mesh = plsc.VectorSubcoreMesh(core_axis_name="c", subcore_axis_name="s")
    fn = pl.kernel(
        functools.partial(_sc_body, nc, rpw),
        out_type=jax.ShapeDtypeStruct((ROWS, D), jnp.float32),
        mesh=mesh,
        scratch_types=[
            pltpu.VMEM((rpw, HP), jnp.int32),
            pltpu.VMEM((rpw, HP), jnp.int32),
            pltpu.VMEM((rpw, HP), jnp.float32),
            pltpu.VMEM((rpw, HP), jnp.float32),
            pltpu.VMEM((HP, DH), jnp.float32),
            pltpu.VMEM((HP, DH), jnp.float32),
            pltpu.VMEM((D,), jnp.float32),
            pltpu.SemaphoreType.DMA,
            pltpu.SemaphoreType.DMA,
        ],
        compiler_params=pltpu.CompilerParams(use_tc_tiling_on_sc=False),
    )
    return fn(vtab, i0, i1, c0, c1)


def _stage_c_body(s_ref, x_ref, wi_ref, bi_ref, wo_ref, bo_ref, out_ref):
    y = (jnp.dot(s_ref[0], wi_ref[...], preferred_element_type=jnp.float32)
         + bi_ref[...] + x_ref[0])
    out_ref[0] = (jnp.dot(y, wo_ref[...], preferred_element_type=jnp.float32)
                  + bo_ref[...])


def _stage_c(sampled, x, W_op_i, b_op_i, W_op_o, b_op_o):
    grid = (B, L // TL)
    return pl.pallas_call(
        _stage_c_body,
        grid=grid,
        in_specs=[
            pl.BlockSpec((1, TL, D), lambda b, i: (b, i, 0)),
            pl.BlockSpec((1, TL, D), lambda b, i: (b, i, 0)),
            pl.BlockSpec((D, D), lambda b, i: (0, 0)),
            pl.BlockSpec((1, D), lambda b, i: (0, 0)),
            pl.BlockSpec((D, D), lambda b, i: (0, 0)),
            pl.BlockSpec((1, D), lambda b, i: (0, 0)),
        ],
        out_specs=pl.BlockSpec((1, TL, D), lambda b, i: (b, i, 0)),
        out_shape=jax.ShapeDtypeStruct((B, L, D), jnp.float32),
    )(sampled, x, W_op_i, b_op_i.reshape(1, D), W_op_o, b_op_o.reshape(1, D))


def kernel(x, W_vp_o, b_vp_o, W_so, b_so, W_aw, b_aw, W_vp_i, b_vp_i,
           W_op_i, b_op_i, W_op_o, b_op_o):
    Wv, bv = _fold(W_vp_o, W_vp_i, b_vp_o, b_vp_i)
    W_cat = jnp.concatenate([Wv, W_so[:, 0::2], W_so[:, 1::2], W_aw], axis=1)
    b_cat = jnp.concatenate(
        [bv, b_so[0::2][None], b_so[1::2][None], b_aw[None]], axis=1)
    gi = jnp.arange(HP) // P
    G = (gi[:, None] == gi[None, :]).astype(jnp.float32)
    value, i0, i1, c0, c1 = _stage_a(x, W_cat, b_cat, G)
    vtab = value.reshape(B * L * H, DH)
    sampled = _stage_b(vtab, i0.reshape(ROWS, HP), i1.reshape(ROWS, HP),
                       c0.reshape(ROWS, HP), c1.reshape(ROWS, HP))
    return _stage_c(sampled.reshape(B, L, D), x, W_op_i, b_op_i,
                    W_op_o, b_op_o)
